# baseline (speedup 1.0000x reference)
import jax
import jax.numpy as jnp
from jax import lax
from jax.experimental import pallas as pl
from jax.experimental.pallas import tpu as pltpu
from jax.experimental.pallas import tpu_sc as plsc

NU = 25000
NI = 25000
E = 400000
D = 128

NP = 50176
NCORE = 2
NSUB = 16
NT = NCORE * NSUB
TROWS = NP // NT
PR = TROWS // 2
ED = 2 * E
EDP = 800896
PAD_SRC = NP - 1
DEGW = 1600

_mesh = plsc.VectorSubcoreMesh(core_axis_name="c", subcore_axis_name="s")


def _rangemask(x, n):
    return jnp.minimum(jnp.maximum(jnp.minimum(x, n - 1 - x) + 1, 0), 1)


def _deg_body(listD_hbm, a0_hbm, nb_hbm, deg_hbm, *, deg_v, didx_v, cw_v):
    c = lax.axis_index("c")
    s = lax.axis_index("s")
    wid = c * NSUB + s
    iot = lax.iota(jnp.int32, 16)
    zf = jnp.zeros((16,), jnp.float32)

    def _z(k, _):
        deg_v[pl.ds(k * 16, 16)] = zf
        return 0
    lax.fori_loop(0, DEGW // 16, _z, 0)

    pltpu.sync_copy(a0_hbm.at[pl.ds(pl.multiple_of(wid * 16, 8), 16)],
                    cw_v)
    a0 = cw_v[pl.ds(0, 16)][0]
    pltpu.sync_copy(nb_hbm.at[pl.ds(pl.multiple_of(wid * 16, 8), 16)],
                    cw_v)
    nb = cw_v[pl.ds(0, 16)][0]

    def _batch(i, _):
        pltpu.sync_copy(
            listD_hbm.at[pl.ds(pl.multiple_of(a0 + i * 64, 8), 64)],
            didx_v)
        for g4 in range(4):
            dvec = didx_v[pl.ds(g4 * 16, 16)] - wid * TROWS
            inr = _rangemask(dvec, TROWS)
            dvec = inr * dvec + (1 - inr) * (TROWS + 15)
            for j2 in range(16):
                dstl = dvec[j2]
                oh = (1 - jnp.minimum(
                    jnp.maximum(iot - (dstl & 15),
                                (dstl & 15) - iot), 1)).astype(jnp.float32)
                plsc.addupdate(deg_v.at[pl.ds((dstl >> 4) * 16, 16)], oh)
        return 0
    lax.fori_loop(0, nb, _batch, 0)

    pltpu.sync_copy(deg_v,
                    deg_hbm.at[pl.ds(pl.multiple_of(wid * DEGW, 8), DEGW)])


_deg_kernel = pl.kernel(
    _deg_body,
    out_type=jax.ShapeDtypeStruct((NT * DEGW,), jnp.float32),
    mesh=_mesh,
    scratch_types=dict(
        deg_v=pltpu.VMEM((DEGW,), jnp.float32),
        didx_v=pltpu.VMEM((64,), jnp.int32),
        cw_v=pltpu.VMEM((16,), jnp.int32),
    ),
)


def _make_layer(mode):
    def _body(*refs, acc_v, rows_v, sidx_v, didx_v, s2_v, d2_v, deg_v,
              cw_v, wb1_v, wb2_v, wb3_v, sem):
        if mode == 1:
            (emb_hbm, listS_hbm, listD_hbm, a0_hbm, nb_hbm, deg_hbm,
             out_hbm, acco_hbm) = refs
            acci_hbm = None
        elif mode == 2:
            (emb_hbm, acci_hbm, listS_hbm, listD_hbm, a0_hbm, nb_hbm,
             deg_hbm, out_hbm, acco_hbm) = refs
        else:
            (emb_hbm, acci_hbm, listS_hbm, listD_hbm, a0_hbm, nb_hbm,
             deg_hbm, out_hbm) = refs
            acco_hbm = None
        c = lax.axis_index("c")
        s = lax.axis_index("s")
        wid = c * NSUB + s
        zf = jnp.zeros((16,), jnp.float32)

        pltpu.sync_copy(a0_hbm.at[pl.ds(pl.multiple_of(wid * 16, 8), 16)],
                        cw_v)
        a0 = cw_v[pl.ds(0, 16)][0]
        pltpu.sync_copy(nb_hbm.at[pl.ds(pl.multiple_of(wid * 16, 8), 16)],
                        cw_v)
        nb = cw_v[pl.ds(0, 16)][0]

        for p in range(2):
            lo = wid * TROWS + p * PR

            def _z(r, _):
                for k in range(8):
                    acc_v[r, pl.ds(k * 16, 16)] = zf
                return 0
            lax.fori_loop(0, PR + 1, _z, 0)

            pltpu.sync_copy(
                deg_hbm.at[pl.ds(pl.multiple_of(wid * DEGW + p * PR, 8), PR)],
                deg_v)

            def _batch(i, _):
                pltpu.sync_copy(
                    listS_hbm.at[pl.ds(pl.multiple_of(a0 + i * 64, 8), 64)],
                    sidx_v)
                pltpu.sync_copy(
                    listD_hbm.at[pl.ds(pl.multiple_of(a0 + i * 64, 8), 64)],
                    didx_v)
                for k in range(4):
                    dg = didx_v[pl.ds(k * 16, 16)] - lo
                    m = _rangemask(dg, PR)
                    d2_v[pl.ds(k * 16, 16)] = m * dg + (1 - m) * PR
                    s2_v[pl.ds(k * 16, 16)] = (
                        m * sidx_v[pl.ds(k * 16, 16)] + (1 - m) * PAD_SRC)
                pltpu.async_copy(emb_hbm.at[s2_v], rows_v, sem).wait()

                for g4 in range(4):
                    dvec = d2_v[pl.ds(g4 * 16, 16)]
                    for j2 in range(16):
                        dstl = dvec[j2]
                        g = g4 * 16 + j2
                        for k in range(8):
                            plsc.addupdate(
                                acc_v.at[dstl, pl.ds(k * 16, 16)],
                                rows_v[g, pl.ds(k * 16, 16)])
                return 0
            lax.fori_loop(0, nb, _batch, 0)

            def _wb(blk, _):
                r0 = blk * 16
                gl0 = lo + r0
                pltpu.sync_copy(
                    emb_hbm.at[pl.ds(pl.multiple_of(gl0, 8), 16)], wb1_v)
                if mode != 1:
                    pltpu.sync_copy(
                        acci_hbm.at[pl.ds(pl.multiple_of(gl0, 8), 16)],
                        wb3_v)
                dv16 = deg_v[pl.ds(pl.multiple_of(r0, 8), 16)]

                for r in range(16):
                    dsp = dv16[r] + 1.0
                    for k in range(8):
                        sl = pl.ds(k * 16, 16)
                        o = (wb1_v[r, sl] + acc_v[r0 + r, sl]) / dsp
                        if mode == 1:
                            wb2_v[r, sl] = o
                            wb3_v[r, sl] = wb1_v[r, sl] + o
                        elif mode == 2:
                            wb2_v[r, sl] = o
                            wb3_v[r, sl] = wb3_v[r, sl] + o
                        else:
                            wb2_v[r, sl] = (wb3_v[r, sl] + o) * 0.25

                pltpu.sync_copy(
                    wb2_v, out_hbm.at[pl.ds(pl.multiple_of(gl0, 8), 16)])
                if mode != 3:
                    pltpu.sync_copy(
                        wb3_v, acco_hbm.at[pl.ds(pl.multiple_of(gl0, 8), 16)])
                return 0
            lax.fori_loop(0, PR // 16, _wb, 0)

    if mode == 3:
        outs = jax.ShapeDtypeStruct((NP, D), jnp.float32)
    else:
        outs = (jax.ShapeDtypeStruct((NP, D), jnp.float32),
                jax.ShapeDtypeStruct((NP, D), jnp.float32))
    return pl.kernel(
        _body,
        out_type=outs,
        mesh=_mesh,
        scratch_types=dict(
            acc_v=pltpu.VMEM((PR + 1, D), jnp.float32),
            rows_v=pltpu.VMEM((64, D), jnp.float32),
            sidx_v=pltpu.VMEM((64,), jnp.int32),
            didx_v=pltpu.VMEM((64,), jnp.int32),
            s2_v=pltpu.VMEM((64,), jnp.int32),
            d2_v=pltpu.VMEM((64,), jnp.int32),
            deg_v=pltpu.VMEM((PR,), jnp.float32),
            cw_v=pltpu.VMEM((16,), jnp.int32),
            wb1_v=pltpu.VMEM((16, D), jnp.float32),
            wb2_v=pltpu.VMEM((16, D), jnp.float32),
            wb3_v=pltpu.VMEM((16, D), jnp.float32),
            sem=pltpu.SemaphoreType.DMA,
        ),
    )


_layer1 = _make_layer(1)
_layer2 = _make_layer(2)
_layer3 = _make_layer(3)


def kernel(edge_index, user_embedding, item_embedding):
    eu = edge_index[0].astype(jnp.int32)
    ev = edge_index[1].astype(jnp.int32)

    dst = jnp.concatenate([eu, ev + NU])
    src = jnp.concatenate([ev + NU, eu])

    own = dst // TROWS
    order = jnp.argsort(own)
    dst_s = dst[order]
    src_s = src[order]
    npad = EDP - ED
    dst_s = jnp.concatenate([dst_s, jnp.full((npad,), PAD_SRC, jnp.int32)])
    src_s = jnp.concatenate([src_s, jnp.full((npad,), PAD_SRC, jnp.int32)])
    cnts = jnp.bincount(own, length=NT).astype(jnp.int32)
    ends = jnp.cumsum(cnts).astype(jnp.int32)
    starts = ends - cnts
    a0 = (starts // 64) * 64
    nb = (starts - a0 + cnts + 63) // 64
    a0 = jnp.repeat(a0, 16)
    nb = jnp.repeat(nb, 16)

    e0 = jnp.concatenate([
        user_embedding.astype(jnp.float32),
        item_embedding.astype(jnp.float32),
        jnp.zeros((NP - NU - NI, D), jnp.float32),
    ])

    deg = _deg_kernel(dst_s, a0, nb)
    e1, acc1 = _layer1(e0, src_s, dst_s, a0, nb, deg)
    e2, acc2 = _layer2(e1, acc1, src_s, dst_s, a0, nb, deg)
    fin = _layer3(e2, acc2, src_s, dst_s, a0, nb, deg)
    return fin[:NU], fin[NU:NU + NI]
